# Initial kernel scaffold; baseline (speedup 1.0000x reference)
import functools

import jax
import jax.numpy as jnp
from jax import lax
from jax.experimental import pallas as pl
from jax.experimental.pallas import tpu as pltpu
from jax.experimental.pallas import tpu_sc as plsc

N = 10000
E = 320000
D = 128

NC = 2
NS = 16
NW = NC * NS
E_PER_W = E // NW
CHUNK = 80
NCHUNK = E_PER_W // CHUNK
ROWS_PER_TILE = N // NS
CW = 16


def _agg_body(h_hbm, src_hbm, dst_hbm, zeros_hbm, zeros16_hbm, ones_hbm,
              outp_hbm, outc_hbm, idx_v, dst_v, rows_v, ones_v, acc_sh,
              cnt_sh, sem):
    c = lax.axis_index("c")
    s = lax.axis_index("s")
    wid = s * NC + c

    pltpu.sync_copy(zeros_hbm.at[pl.ds(0, ROWS_PER_TILE)],
                    acc_sh.at[pl.ds(s * ROWS_PER_TILE, ROWS_PER_TILE)])
    pltpu.sync_copy(zeros16_hbm.at[pl.ds(0, ROWS_PER_TILE)],
                    cnt_sh.at[pl.ds(s * ROWS_PER_TILE, ROWS_PER_TILE)])
    pltpu.sync_copy(ones_hbm, ones_v)
    plsc.subcore_barrier()

    base = wid * E_PER_W

    @pl.loop(0, NCHUNK)
    def _(i):
        start = base + i * CHUNK
        pltpu.sync_copy(src_hbm.at[pl.ds(start, CHUNK)], idx_v)
        pltpu.sync_copy(dst_hbm.at[pl.ds(start, CHUNK)], dst_v)
        pltpu.async_copy(h_hbm.at[idx_v], rows_v, sem).wait()
        pltpu.sync_copy(rows_v, acc_sh.at[dst_v], add=True)
        pltpu.sync_copy(ones_v, cnt_sh.at[dst_v], add=True)

    plsc.subcore_barrier()

    row0 = s * ROWS_PER_TILE
    out0 = c * N + row0
    pltpu.sync_copy(acc_sh.at[pl.ds(row0, ROWS_PER_TILE)],
                    outp_hbm.at[pl.ds(out0, ROWS_PER_TILE)])
    pltpu.sync_copy(cnt_sh.at[pl.ds(row0, ROWS_PER_TILE)],
                    outc_hbm.at[pl.ds(out0, ROWS_PER_TILE)])


_agg = pl.kernel(
    _agg_body,
    out_type=(
        jax.ShapeDtypeStruct((NC * N, D), jnp.float32),
        jax.ShapeDtypeStruct((NC * N, CW), jnp.float32),
    ),
    mesh=plsc.VectorSubcoreMesh(core_axis_name="c", subcore_axis_name="s"),
    scratch_types=[
        pltpu.VMEM((CHUNK,), jnp.int32),
        pltpu.VMEM((CHUNK,), jnp.int32),
        pltpu.VMEM((CHUNK, D), jnp.float32),
        pltpu.VMEM((CHUNK, CW), jnp.float32),
        pltpu.VMEM_SHARED((N, D), jnp.float32),
        pltpu.VMEM_SHARED((N, CW), jnp.float32),
        pltpu.SemaphoreType.DMA,
    ],
)


BLK = 500


def _sage_tc_body(relu, p_ref, cnt_ref, h_ref, wl_ref, wr_ref, b_ref, o_ref):
    psum = p_ref[0] + p_ref[1]
    cnt = cnt_ref[0, :, 0:1] + cnt_ref[1, :, 0:1]
    mean = psum / jnp.maximum(cnt, 1.0)
    acc = (jnp.dot(mean, wl_ref[...], preferred_element_type=jnp.float32)
           + jnp.dot(h_ref[...], wr_ref[...], preferred_element_type=jnp.float32)
           + b_ref[...])
    o_ref[...] = jnp.maximum(acc, 0.0) if relu else acc


def _sage_tc(p, cnt, h, wl, wr, b, relu):
    grid = (N // BLK,)
    return pl.pallas_call(
        functools.partial(_sage_tc_body, relu),
        grid=grid,
        in_specs=[
            pl.BlockSpec((NC, BLK, D), lambda i: (0, i, 0)),
            pl.BlockSpec((NC, BLK, CW), lambda i: (0, i, 0)),
            pl.BlockSpec((BLK, D), lambda i: (i, 0)),
            pl.BlockSpec((D, D), lambda i: (0, 0)),
            pl.BlockSpec((D, D), lambda i: (0, 0)),
            pl.BlockSpec((D,), lambda i: (0,)),
        ],
        out_specs=pl.BlockSpec((BLK, D), lambda i: (i, 0)),
        out_shape=jax.ShapeDtypeStruct((N, D), jnp.float32),
    )(p, cnt, h, wl, wr, b)


def kernel(x, edge_index, W1l, W1r, b1, W2l, W2r, b2, W3l, W3r, b3):
    src = edge_index[0]
    dst = edge_index[1]
    zeros = jnp.zeros((ROWS_PER_TILE, D), jnp.float32)
    zeros16 = jnp.zeros((ROWS_PER_TILE, CW), jnp.float32)
    ones = jnp.ones((CHUNK, CW), jnp.float32)

    def layer(h, wl, wr, b, relu):
        p, cntp = _agg(h, src, dst, zeros, zeros16, ones)
        p = p.reshape(NC, N, D)
        cntp = cntp.reshape(NC, N, CW)
        return _sage_tc(p, cntp, h, wl, wr, b, relu)

    h1 = layer(x, W1l, W1r, b1, True)
    x_emb = layer(h1, W2l, W2r, b2, True)
    out = layer(x_emb, W3l, W3r, b3, False)
    return (out, x_emb)


# speedup vs baseline: 4.6420x; 4.6420x over previous
import functools

import jax
import jax.numpy as jnp
from jax import lax
from jax.experimental import pallas as pl
from jax.experimental.pallas import tpu as pltpu
from jax.experimental.pallas import tpu_sc as plsc

N = 10000
E = 320000
D = 128

NC = 2
NS = 16
NW = NC * NS
E_PER_W = E // NW
CHUNK = 80
NCHUNK = E_PER_W // CHUNK
N_PAD = 10240
ROWS_PER_TILE = N_PAD // NS


def _fill_rowidx(zrow_v, base_row):
    for k in range(CHUNK // 16):
        zrow_v[pl.ds(16 * k, 16)] = (
            base_row + 16 * k + lax.iota(jnp.int32, 16))


def _seg_body(gather, h_hbm, src_hbm, dst_hbm, zeros_hbm,
              outp_hbm, idx_v, dst_v, rows_v, zrow_v, acc_sh, sem):
    c = lax.axis_index("c")
    s = lax.axis_index("s")
    wid = s * NC + c
    row0 = pl.multiple_of(s * ROWS_PER_TILE, 8)

    pltpu.sync_copy(zeros_hbm, rows_v)
    for J in range(ROWS_PER_TILE // CHUNK):
        _fill_rowidx(zrow_v, row0 + J * CHUNK)
        pltpu.sync_copy(rows_v, acc_sh.at[zrow_v])
    if not gather:
        pltpu.sync_copy(h_hbm, rows_v)
    plsc.subcore_barrier()

    base = wid * E_PER_W

    def body(i, carry):
        start = pl.multiple_of(base + i * CHUNK, 8)
        pltpu.sync_copy(dst_hbm.at[pl.ds(start, CHUNK)], dst_v)
        if gather:
            pltpu.sync_copy(src_hbm.at[pl.ds(start, CHUNK)], idx_v)
            pltpu.async_copy(h_hbm.at[idx_v], rows_v, sem).wait()
        pltpu.sync_copy(rows_v, acc_sh.at[dst_v], add=True)
        return carry

    lax.fori_loop(0, NCHUNK, body, 0)
    plsc.subcore_barrier()

    for J in range(ROWS_PER_TILE // CHUNK):
        _fill_rowidx(zrow_v, row0 + J * CHUNK)
        out0 = pl.multiple_of(c * N_PAD + row0 + J * CHUNK, 8)
        pltpu.async_copy(acc_sh.at[zrow_v], rows_v, sem).wait()
        pltpu.sync_copy(rows_v, outp_hbm.at[pl.ds(out0, CHUNK)])


def _make_seg(gather):
    return pl.kernel(
        functools.partial(_seg_body, gather),
        out_type=jax.ShapeDtypeStruct((NC * N_PAD, D), jnp.float32),
        mesh=plsc.VectorSubcoreMesh(core_axis_name="c", subcore_axis_name="s"),
        scratch_types=[
            pltpu.VMEM((CHUNK,), jnp.int32),
            pltpu.VMEM((CHUNK,), jnp.int32),
            pltpu.VMEM((CHUNK, D), jnp.float32),
            pltpu.VMEM((CHUNK,), jnp.int32),
            pltpu.VMEM_SHARED((N_PAD, D), jnp.float32),
            pltpu.SemaphoreType.DMA,
        ],
    )


_agg = _make_seg(True)
_cnt = _make_seg(False)


BLK = 400


def _sage_tc_body(relu, p_ref, cnt_ref, h_ref, wl_ref, wr_ref, b_ref, o_ref):
    psum = p_ref[0] + p_ref[1]
    cnt = cnt_ref[0, :, 0:1] + cnt_ref[1, :, 0:1]
    mean = psum / jnp.maximum(cnt, 1.0)
    acc = (jnp.dot(mean, wl_ref[...], preferred_element_type=jnp.float32)
           + jnp.dot(h_ref[...], wr_ref[...], preferred_element_type=jnp.float32)
           + b_ref[...])
    o_ref[...] = jnp.maximum(acc, 0.0) if relu else acc


def _sage_tc(p, cnt, h, wl, wr, b, relu):
    grid = (N // BLK,)
    return pl.pallas_call(
        functools.partial(_sage_tc_body, relu),
        grid=grid,
        in_specs=[
            pl.BlockSpec((NC, BLK, D), lambda i: (0, i, 0)),
            pl.BlockSpec((NC, BLK, D), lambda i: (0, i, 0)),
            pl.BlockSpec((BLK, D), lambda i: (i, 0)),
            pl.BlockSpec((D, D), lambda i: (0, 0)),
            pl.BlockSpec((D, D), lambda i: (0, 0)),
            pl.BlockSpec((1, D), lambda i: (0, 0)),
        ],
        out_specs=pl.BlockSpec((BLK, D), lambda i: (i, 0)),
        out_shape=jax.ShapeDtypeStruct((N, D), jnp.float32),
    )(p, cnt, h, wl, wr, b)


def kernel(x, edge_index, W1l, W1r, b1, W2l, W2r, b2, W3l, W3r, b3):
    src = edge_index[0]
    dst = edge_index[1]
    zeros = jnp.zeros((CHUNK, D), jnp.float32)
    ones = jnp.ones((CHUNK, D), jnp.float32)

    cntp = _cnt(ones, src, dst, zeros).reshape(NC, N_PAD, D)

    def layer(h, wl, wr, b, relu):
        p = _agg(h, src, dst, zeros).reshape(NC, N_PAD, D)
        return _sage_tc(p, cntp, h, wl, wr, b.reshape(1, D), relu)

    h1 = layer(x, W1l, W1r, b1, True)
    x_emb = layer(h1, W2l, W2r, b2, True)
    out = layer(x_emb, W3l, W3r, b3, False)
    return (out, x_emb)


# speedup vs baseline: 5.9657x; 1.2852x over previous
import functools

import jax
import jax.numpy as jnp
from jax import lax
from jax.experimental import pallas as pl
from jax.experimental.pallas import tpu as pltpu
from jax.experimental.pallas import tpu_sc as plsc

N = 10000
E = 320000
D = 128

NC = 2
NS = 16
NW = NC * NS
E_PER_W = E // NW
CHUNK = 80
NCHUNK = E_PER_W // CHUNK
N_PAD = 10240
ROWS_PER_TILE = N_PAD // NS


def _fill_rowidx(zrow_v, base_row):
    for k in range(CHUNK // 16):
        zrow_v[pl.ds(16 * k, 16)] = (
            base_row + 16 * k + lax.iota(jnp.int32, 16))


def _seg_body(gather, h_hbm, src_hbm, dst_hbm, zeros_hbm,
              outp_hbm, idx_v, idx2_v, dst_v, dst2_v, rows_v, rows2_v,
              zrow_v, acc_sh, sem, sem2):
    c = lax.axis_index("c")
    s = lax.axis_index("s")
    wid = s * NC + c
    row0 = pl.multiple_of(s * ROWS_PER_TILE, 8)

    pltpu.sync_copy(zeros_hbm, rows_v)
    for J in range(ROWS_PER_TILE // CHUNK):
        _fill_rowidx(zrow_v, row0 + J * CHUNK)
        pltpu.sync_copy(rows_v, acc_sh.at[zrow_v])
    if not gather:
        pltpu.sync_copy(h_hbm, rows_v)
        pltpu.sync_copy(h_hbm, rows2_v)
    plsc.subcore_barrier()

    base = wid * E_PER_W

    def pair_body(i, carry):
        st_a = pl.multiple_of(base + (2 * i) * CHUNK, 8)
        st_b = pl.multiple_of(base + (2 * i + 1) * CHUNK, 8)
        pltpu.sync_copy(dst_hbm.at[pl.ds(st_a, CHUNK)], dst_v)
        pltpu.sync_copy(dst_hbm.at[pl.ds(st_b, CHUNK)], dst2_v)
        if gather:
            pltpu.sync_copy(src_hbm.at[pl.ds(st_a, CHUNK)], idx_v)
            cp_a = pltpu.async_copy(h_hbm.at[idx_v], rows_v, sem)
            pltpu.sync_copy(src_hbm.at[pl.ds(st_b, CHUNK)], idx2_v)
            cp_b = pltpu.async_copy(h_hbm.at[idx2_v], rows2_v, sem2)
            cp_a.wait()
        pltpu.sync_copy(rows_v, acc_sh.at[dst_v], add=True)
        if gather:
            cp_b.wait()
        pltpu.sync_copy(rows2_v, acc_sh.at[dst2_v], add=True)
        return carry

    lax.fori_loop(0, NCHUNK // 2, pair_body, 0)
    st_z = pl.multiple_of(base + (NCHUNK - 1) * CHUNK, 8)
    pltpu.sync_copy(dst_hbm.at[pl.ds(st_z, CHUNK)], dst_v)
    if gather:
        pltpu.sync_copy(src_hbm.at[pl.ds(st_z, CHUNK)], idx_v)
        pltpu.async_copy(h_hbm.at[idx_v], rows_v, sem).wait()
    pltpu.sync_copy(rows_v, acc_sh.at[dst_v], add=True)
    plsc.subcore_barrier()

    for J in range(ROWS_PER_TILE // CHUNK):
        _fill_rowidx(zrow_v, row0 + J * CHUNK)
        out0 = pl.multiple_of(c * N_PAD + row0 + J * CHUNK, 8)
        pltpu.async_copy(acc_sh.at[zrow_v], rows_v, sem).wait()
        pltpu.sync_copy(rows_v, outp_hbm.at[pl.ds(out0, CHUNK)])


def _make_seg(gather):
    return pl.kernel(
        functools.partial(_seg_body, gather),
        out_type=jax.ShapeDtypeStruct((NC * N_PAD, D), jnp.float32),
        mesh=plsc.VectorSubcoreMesh(core_axis_name="c", subcore_axis_name="s"),
        scratch_types=[
            pltpu.VMEM((CHUNK,), jnp.int32),
            pltpu.VMEM((CHUNK,), jnp.int32),
            pltpu.VMEM((CHUNK,), jnp.int32),
            pltpu.VMEM((CHUNK,), jnp.int32),
            pltpu.VMEM((CHUNK, D), jnp.float32),
            pltpu.VMEM((CHUNK, D), jnp.float32),
            pltpu.VMEM((CHUNK,), jnp.int32),
            pltpu.VMEM_SHARED((N_PAD, D), jnp.float32),
            pltpu.SemaphoreType.DMA,
            pltpu.SemaphoreType.DMA,
        ],
    )


_agg = _make_seg(True)
_cnt = _make_seg(False)


BLK = 400


def _sage_tc_body(relu, p_ref, cnt_ref, h_ref, wl_ref, wr_ref, b_ref, o_ref):
    psum = p_ref[0] + p_ref[1]
    cnt = cnt_ref[0, :, 0:1] + cnt_ref[1, :, 0:1]
    mean = psum / jnp.maximum(cnt, 1.0)
    acc = (jnp.dot(mean, wl_ref[...], preferred_element_type=jnp.float32)
           + jnp.dot(h_ref[...], wr_ref[...], preferred_element_type=jnp.float32)
           + b_ref[...])
    o_ref[...] = jnp.maximum(acc, 0.0) if relu else acc


def _sage_tc(p, cnt, h, wl, wr, b, relu):
    grid = (N // BLK,)
    return pl.pallas_call(
        functools.partial(_sage_tc_body, relu),
        grid=grid,
        in_specs=[
            pl.BlockSpec((NC, BLK, D), lambda i: (0, i, 0)),
            pl.BlockSpec((NC, BLK, D), lambda i: (0, i, 0)),
            pl.BlockSpec((BLK, D), lambda i: (i, 0)),
            pl.BlockSpec((D, D), lambda i: (0, 0)),
            pl.BlockSpec((D, D), lambda i: (0, 0)),
            pl.BlockSpec((1, D), lambda i: (0, 0)),
        ],
        out_specs=pl.BlockSpec((BLK, D), lambda i: (i, 0)),
        out_shape=jax.ShapeDtypeStruct((N, D), jnp.float32),
    )(p, cnt, h, wl, wr, b)


def kernel(x, edge_index, W1l, W1r, b1, W2l, W2r, b2, W3l, W3r, b3):
    src = edge_index[0]
    dst = edge_index[1]
    zeros = jnp.zeros((CHUNK, D), jnp.float32)
    ones = jnp.ones((CHUNK, D), jnp.float32)

    cntp = _cnt(ones, src, dst, zeros).reshape(NC, N_PAD, D)

    def layer(h, wl, wr, b, relu):
        p = _agg(h, src, dst, zeros).reshape(NC, N_PAD, D)
        return _sage_tc(p, cntp, h, wl, wr, b.reshape(1, D), relu)

    h1 = layer(x, W1l, W1r, b1, True)
    x_emb = layer(h1, W2l, W2r, b2, True)
    out = layer(x_emb, W3l, W3r, b3, False)
    return (out, x_emb)


# speedup vs baseline: 6.8073x; 1.1411x over previous
import functools

import jax
import jax.numpy as jnp
from jax import lax
from jax.experimental import pallas as pl
from jax.experimental.pallas import tpu as pltpu
from jax.experimental.pallas import tpu_sc as plsc

N = 10000
E = 320000
D = 128

NC = 2
NS = 16
NW = NC * NS
E_PER_W = E // NW
CHUNK = 80
NCHUNK = E_PER_W // CHUNK
N_PAD = 10240
ROWS_PER_TILE = N_PAD // NS


def _fill_rowidx(zrow_v, base_row):
    for k in range(CHUNK // 16):
        zrow_v[pl.ds(16 * k, 16)] = (
            base_row + 16 * k + lax.iota(jnp.int32, 16))


DEPTH = 4


def _seg_body(gather, h_hbm, src_hbm, dst_hbm, zeros_hbm, outp_hbm, *scr):
    idx_bufs = scr[0:DEPTH]
    dst_bufs = scr[DEPTH:2 * DEPTH]
    rows_bufs = scr[2 * DEPTH:3 * DEPTH]
    zrow_v = scr[3 * DEPTH]
    acc_sh = scr[3 * DEPTH + 1]
    sems = scr[3 * DEPTH + 2:]

    c = lax.axis_index("c")
    s = lax.axis_index("s")
    wid = s * NC + c
    row0 = pl.multiple_of(s * ROWS_PER_TILE, 8)

    pltpu.sync_copy(zeros_hbm, rows_bufs[0])
    for J in range(ROWS_PER_TILE // CHUNK):
        _fill_rowidx(zrow_v, row0 + J * CHUNK)
        pltpu.sync_copy(rows_bufs[0], acc_sh.at[zrow_v])
    if not gather:
        for k in range(DEPTH):
            pltpu.sync_copy(h_hbm, rows_bufs[k])
    plsc.subcore_barrier()

    base = wid * E_PER_W

    def body(i, carry):
        cps = []
        for k in range(DEPTH):
            st = pl.multiple_of(base + (DEPTH * i + k) * CHUNK, 8)
            pltpu.sync_copy(dst_hbm.at[pl.ds(st, CHUNK)], dst_bufs[k])
            if gather:
                pltpu.sync_copy(src_hbm.at[pl.ds(st, CHUNK)], idx_bufs[k])
                cps.append(pltpu.async_copy(
                    h_hbm.at[idx_bufs[k]], rows_bufs[k], sems[k]))
        for k in range(DEPTH):
            if gather:
                cps[k].wait()
            pltpu.sync_copy(rows_bufs[k], acc_sh.at[dst_bufs[k]], add=True)
        return carry

    lax.fori_loop(0, NCHUNK // DEPTH, body, 0)
    st_z = pl.multiple_of(base + (NCHUNK - 1) * CHUNK, 8)
    pltpu.sync_copy(dst_hbm.at[pl.ds(st_z, CHUNK)], dst_bufs[0])
    if gather:
        pltpu.sync_copy(src_hbm.at[pl.ds(st_z, CHUNK)], idx_bufs[0])
        pltpu.async_copy(h_hbm.at[idx_bufs[0]], rows_bufs[0], sems[0]).wait()
    pltpu.sync_copy(rows_bufs[0], acc_sh.at[dst_bufs[0]], add=True)
    plsc.subcore_barrier()

    for J in range(ROWS_PER_TILE // CHUNK):
        _fill_rowidx(zrow_v, row0 + J * CHUNK)
        out0 = pl.multiple_of(c * N_PAD + row0 + J * CHUNK, 8)
        pltpu.async_copy(acc_sh.at[zrow_v], rows_bufs[0], sems[0]).wait()
        pltpu.sync_copy(rows_bufs[0], outp_hbm.at[pl.ds(out0, CHUNK)])


def _make_seg(gather):
    return pl.kernel(
        functools.partial(_seg_body, gather),
        out_type=jax.ShapeDtypeStruct((NC * N_PAD, D), jnp.float32),
        mesh=plsc.VectorSubcoreMesh(core_axis_name="c", subcore_axis_name="s"),
        scratch_types=(
            [pltpu.VMEM((CHUNK,), jnp.int32) for _ in range(DEPTH)]
            + [pltpu.VMEM((CHUNK,), jnp.int32) for _ in range(DEPTH)]
            + [pltpu.VMEM((CHUNK, D), jnp.float32) for _ in range(DEPTH)]
            + [pltpu.VMEM((CHUNK,), jnp.int32)]
            + [pltpu.VMEM_SHARED((N_PAD, D), jnp.float32)]
            + [pltpu.SemaphoreType.DMA for _ in range(DEPTH)]
        ),
    )


_agg = _make_seg(True)
_cnt = _make_seg(False)


BLK = 400


def _sage_tc_body(relu, p_ref, cnt_ref, h_ref, wl_ref, wr_ref, b_ref, o_ref):
    psum = p_ref[0] + p_ref[1]
    cnt = cnt_ref[0, :, 0:1] + cnt_ref[1, :, 0:1]
    mean = psum / jnp.maximum(cnt, 1.0)
    acc = (jnp.dot(mean, wl_ref[...], preferred_element_type=jnp.float32)
           + jnp.dot(h_ref[...], wr_ref[...], preferred_element_type=jnp.float32)
           + b_ref[...])
    o_ref[...] = jnp.maximum(acc, 0.0) if relu else acc


def _sage_tc(p, cnt, h, wl, wr, b, relu):
    grid = (N // BLK,)
    return pl.pallas_call(
        functools.partial(_sage_tc_body, relu),
        grid=grid,
        in_specs=[
            pl.BlockSpec((NC, BLK, D), lambda i: (0, i, 0)),
            pl.BlockSpec((NC, BLK, D), lambda i: (0, i, 0)),
            pl.BlockSpec((BLK, D), lambda i: (i, 0)),
            pl.BlockSpec((D, D), lambda i: (0, 0)),
            pl.BlockSpec((D, D), lambda i: (0, 0)),
            pl.BlockSpec((1, D), lambda i: (0, 0)),
        ],
        out_specs=pl.BlockSpec((BLK, D), lambda i: (i, 0)),
        out_shape=jax.ShapeDtypeStruct((N, D), jnp.float32),
    )(p, cnt, h, wl, wr, b)


def kernel(x, edge_index, W1l, W1r, b1, W2l, W2r, b2, W3l, W3r, b3):
    src = edge_index[0]
    dst = edge_index[1]
    zeros = jnp.zeros((CHUNK, D), jnp.float32)
    ones = jnp.ones((CHUNK, D), jnp.float32)

    cntp = _cnt(ones, src, dst, zeros).reshape(NC, N_PAD, D)

    def layer(h, wl, wr, b, relu):
        p = _agg(h, src, dst, zeros).reshape(NC, N_PAD, D)
        return _sage_tc(p, cntp, h, wl, wr, b.reshape(1, D), relu)

    h1 = layer(x, W1l, W1r, b1, True)
    x_emb = layer(h1, W2l, W2r, b2, True)
    out = layer(x_emb, W3l, W3r, b3, False)
    return (out, x_emb)
